# baseline (speedup 1.0000x reference)
import functools
import math

import jax
import jax.numpy as jnp
from jax import lax
from jax.experimental import pallas as pl
from jax.experimental.pallas import tpu as pltpu
from jax.experimental.pallas import tpu_sc as plsc

B, N, D, C = 16, 2048, 512, 128
NC, NS, L = 2, 16, 16
NW = NC * NS
DC = D // NW

LN2 = 0.6931471805599453
LOG2PI = 1.8378770664093453

_R0 = 0.0008254628229340533
_R1 = 1.415653190432736
_R2 = -0.5687040530057521
_R3 = 0.15270028479752185

_C0 = 2.443438720245439e-06
_C1 = 1.4424535262105997
_C2 = -0.7173127802648079
_C3 = 0.454508492199418
_C4 = -0.2726975648521658
_C5 = 0.1176130840660221
_C6 = -0.024568534745087942


_S0 = _R0 - 127.0
_S1 = _R1 * 2.0 ** -23
_S2 = _R2 * 2.0 ** -46
_S3 = _R3 * 2.0 ** -69


def _log2_biased(x):
    bits = plsc.bitcast(x, jnp.int32)
    ebias = lax.shift_right_logical(bits, 23).astype(jnp.float32)
    zm = (bits & 0x007FFFFF).astype(jnp.float32)
    poly = ((jnp.float32(_S3) * zm + jnp.float32(_S2)) * zm
            + jnp.float32(_S1)) * zm + jnp.float32(_S0)
    return poly + ebias


def _log2_hi(x):
    bits = plsc.bitcast(x, jnp.int32)
    ebias = lax.shift_right_logical(bits, 23).astype(jnp.float32)
    fbits = (bits & 0x007FFFFF) | 0x3F800000
    z = plsc.bitcast(fbits, jnp.float32) - 1.0
    acc = jnp.float32(_C6)
    for c in (_C5, _C4, _C3, _C2, _C1, _C0):
        acc = acc * z + jnp.float32(c)
    return acc + (ebias - 127.0)


CH = N // 2


def _body(mp_hbm, tgt_hbm, pm_hbm, pp_hbm, part_hbm,
          buf0, buf1, tbuf0, tbuf1,
          acc_p, acc_pm, acc_pm2, acc_l2, cnt, pmout, lpsum,
          sem0, sem1, semo):
    cid = lax.axis_index("c")
    sid = lax.axis_index("s")
    wid = sid * NC + cid
    c0 = wid * DC

    bufs = (buf0, buf1)
    tbufs = (tbuf0, tbuf1)
    sems = (sem0, sem1)

    zvec = jnp.zeros((L,), jnp.float32)
    onevec = jnp.ones((L,), jnp.float32)
    iota = lax.iota(jnp.int32, L)

    def fire(b, h):
        pltpu.async_copy(
            mp_hbm.at[b, pl.ds(h * CH, CH), pl.ds(2 * c0, 2 * DC)],
            bufs[h], sems[h])
        pltpu.async_copy(tgt_hbm.at[b, pl.ds(h * CH, CH)], tbufs[h], sems[h])

    def drain(b, h):
        pltpu.make_async_copy(
            mp_hbm.at[b, pl.ds(h * CH, CH), pl.ds(2 * c0, 2 * DC)],
            bufs[h], sems[h]).wait()
        pltpu.make_async_copy(tgt_hbm.at[b, pl.ds(h * CH, CH)],
                              tbufs[h], sems[h]).wait()

    def zero_partial_accs():
        @plsc.parallel_loop(0, C, unroll=4)
        def zloop(c):
            acc_pm[c] = zvec
            acc_pm2[c] = zvec
            acc_l2[c] = zvec

        @plsc.parallel_loop(0, C // L, unroll=2)
        def zcnt(g):
            cnt[pl.ds(g * L, L)] = zvec

    def zero_accp():
        @plsc.parallel_loop(0, C, unroll=4)
        def zp(c):
            acc_p[c] = zvec

    def zero_lpsum():
        @plsc.parallel_loop(0, C // L, unroll=2)
        def zl(g):
            lpsum[pl.ds(g * L, L)] = zvec

    def fire_out(b):
        pltpu.async_copy(acc_p, pp_hbm.at[b, :, pl.ds(c0, DC)], semo)
        pltpu.async_copy(pmout, pm_hbm.at[b, :, pl.ds(c0, DC)], semo)
        pltpu.async_copy(lpsum, part_hbm.at[wid, b], semo)

    def drain_out(b):
        pltpu.make_async_copy(acc_p, pp_hbm.at[b, :, pl.ds(c0, DC)],
                              semo).wait()
        pltpu.make_async_copy(pmout, pm_hbm.at[b, :, pl.ds(c0, DC)],
                              semo).wait()
        pltpu.make_async_copy(lpsum, part_hbm.at[wid, b], semo).wait()

    def do_half(b, h):
        if h == 0:
            fire(b, 1)
        else:
            @pl.when(b < B - 1)
            def _():
                fire(b + 1, 0)

        drain(b, h)
        buf = bufs[h]
        tbuf = tbufs[h]

        @plsc.parallel_loop(0, CH // L, unroll=4)
        def count(g):
            tvec = tbuf[pl.ds(g * L, L)]
            plsc.addupdate_scatter(cnt, [tvec], onevec)

        @plsc.parallel_loop(0, CH, unroll=8)
        def row(n):
            tb = plsc.load_gather(tbuf, [jnp.broadcast_to(n, (L,))])
            raw = buf[n]
            p, m = plsc.unpack(raw, format=plsc.PackFormat.INTERLEAVED,
                               preferred_element_type=jnp.float32)
            pm = p * m
            pmm = pm * m
            l2 = _log2_biased(p)
            plsc.addupdate_scatter(acc_p, [tb, iota], p)
            plsc.addupdate_scatter(acc_pm, [tb, iota], pm)
            plsc.addupdate_scatter(acc_pm2, [tb, iota], pmm)
            plsc.addupdate_scatter(acc_l2, [tb, iota], l2)

    def batch_body(b, carry):
        do_half(b, 0)
        do_half(b, 1)

        @pl.when(b > 0)
        def _():
            pltpu.make_async_copy(pmout, pm_hbm.at[b - 1, :, pl.ds(c0, DC)],
                                  semo).wait()
            pltpu.make_async_copy(lpsum, part_hbm.at[wid, b - 1], semo).wait()

        zero_lpsum()

        @plsc.parallel_loop(0, C, unroll=4)
        def cflush(c):
            s1 = acc_p[c]
            s2 = acc_pm[c]
            s3 = acc_pm2[c]
            sl = acc_l2[c]
            nv = plsc.load_gather(cnt, [jnp.broadcast_to(c, (L,))])
            pmv = s2 / s1
            pmout[c] = pmv
            l2s1 = _log2_hi(s1)
            nm = jnp.maximum(nv, onevec)
            lp = 0.5 * ((1.0 - nm) * jnp.float32(LOG2PI)
                        + jnp.float32(LN2) * (sl - l2s1)
                        + (s1 * pmv * pmv - s3))
            plsc.addupdate_scatter(lpsum, [jnp.broadcast_to(c, (L,))], lp)

        pltpu.async_copy(acc_p, pp_hbm.at[b, :, pl.ds(c0, DC)], semo)
        pltpu.async_copy(pmout, pm_hbm.at[b, :, pl.ds(c0, DC)], semo)
        pltpu.async_copy(lpsum, part_hbm.at[wid, b], semo)
        zero_partial_accs()
        pltpu.make_async_copy(acc_p, pp_hbm.at[b, :, pl.ds(c0, DC)],
                              semo).wait()
        zero_accp()
        return carry

    zero_partial_accs()
    zero_accp()
    zero_lpsum()
    fire(0, 0)
    lax.fori_loop(0, B, batch_body, 0)
    pltpu.make_async_copy(pmout, pm_hbm.at[B - 1, :, pl.ds(c0, DC)],
                          semo).wait()
    pltpu.make_async_copy(lpsum, part_hbm.at[wid, B - 1], semo).wait()


@jax.jit
def kernel(means, precisions, targets):
    mesh = plsc.VectorSubcoreMesh(core_axis_name="c", subcore_axis_name="s",
                                  num_cores=NC, num_subcores=NS)
    k = pl.kernel(
        _body,
        out_type=(
            jax.ShapeDtypeStruct((B, C, D), jnp.float32),
            jax.ShapeDtypeStruct((B, C, D), jnp.float32),
            jax.ShapeDtypeStruct((NW, B, C), jnp.float32),
        ),
        mesh=mesh,
        compiler_params=pltpu.CompilerParams(use_tc_tiling_on_sc=False,
                                             needs_layout_passes=False),
        scratch_types=[
            pltpu.VMEM((CH, 2 * DC), jnp.bfloat16),
            pltpu.VMEM((CH, 2 * DC), jnp.bfloat16),
            pltpu.VMEM((CH,), jnp.int32),
            pltpu.VMEM((CH,), jnp.int32),
            pltpu.VMEM((C, L), jnp.float32),
            pltpu.VMEM((C, L), jnp.float32),
            pltpu.VMEM((C, L), jnp.float32),
            pltpu.VMEM((C, L), jnp.float32),
            pltpu.VMEM((C,), jnp.float32),
            pltpu.VMEM((C, L), jnp.float32),
            pltpu.VMEM((C,), jnp.float32),
            pltpu.SemaphoreType.DMA,
            pltpu.SemaphoreType.DMA,
            pltpu.SemaphoreType.DMA,
        ],
    )
    mp = jnp.stack((precisions, means), axis=-1).astype(jnp.bfloat16)
    mp = mp.reshape(B, N, 2 * D)
    pm, pp, part = k(mp, targets)
    lpn = part.sum(axis=0)
    return (pm, pp, lpn)


# speedup vs baseline: 1.9511x; 1.9511x over previous
import functools
import math

import jax
import jax.numpy as jnp
from jax import lax
from jax.experimental import pallas as pl
from jax.experimental.pallas import tpu as pltpu
from jax.experimental.pallas import tpu_sc as plsc

B, N, D, C = 16, 2048, 512, 128
NC, NS, L = 2, 16, 16
NW = NC * NS
DC = D // NW

LN2 = 0.6931471805599453
LOG2PI = 1.8378770664093453

_R0 = 0.0008254628229340533
_R1 = 1.415653190432736
_R2 = -0.5687040530057521
_R3 = 0.15270028479752185

_C0 = 2.443438720245439e-06
_C1 = 1.4424535262105997
_C2 = -0.7173127802648079
_C3 = 0.454508492199418
_C4 = -0.2726975648521658
_C5 = 0.1176130840660221
_C6 = -0.024568534745087942


_S0 = _R0 - 127.0
_S1 = _R1 * 2.0 ** -23
_S2 = _R2 * 2.0 ** -46
_S3 = _R3 * 2.0 ** -69


def _log2_biased(x):
    bits = plsc.bitcast(x, jnp.int32)
    ebias = lax.shift_right_logical(bits, 23).astype(jnp.float32)
    zm = (bits & 0x007FFFFF).astype(jnp.float32)
    poly = ((jnp.float32(_S3) * zm + jnp.float32(_S2)) * zm
            + jnp.float32(_S1)) * zm + jnp.float32(_S0)
    return poly + ebias


def _log2_hi(x):
    bits = plsc.bitcast(x, jnp.int32)
    ebias = lax.shift_right_logical(bits, 23).astype(jnp.float32)
    fbits = (bits & 0x007FFFFF) | 0x3F800000
    z = plsc.bitcast(fbits, jnp.float32) - 1.0
    acc = jnp.float32(_C6)
    for c in (_C5, _C4, _C3, _C2, _C1, _C0):
        acc = acc * z + jnp.float32(c)
    return acc + (ebias - 127.0)


CH = N // 2


def _body(mp_hbm, tgt_hbm, pm_hbm, pp_hbm, part_hbm,
          buf0, buf1, tbuf0, tbuf1,
          acc_p, acc_pm, acc_pm2, acc_l2, cnt, pmout, lpsum,
          sem0, sem1, semo):
    cid = lax.axis_index("c")
    sid = lax.axis_index("s")
    wid = sid * NC + cid
    c0 = wid * DC

    bufs = (buf0, buf1)
    tbufs = (tbuf0, tbuf1)
    sems = (sem0, sem1)

    zvec = jnp.zeros((L,), jnp.float32)
    onevec = jnp.ones((L,), jnp.float32)
    iota = lax.iota(jnp.int32, L)

    def fire(b, h):
        pltpu.async_copy(
            mp_hbm.at[b, pl.ds(h * CH, CH), pl.ds(c0, DC)],
            bufs[h], sems[h])
        pltpu.async_copy(tgt_hbm.at[b, pl.ds(h * CH, CH)], tbufs[h], sems[h])

    def drain(b, h):
        pltpu.make_async_copy(
            mp_hbm.at[b, pl.ds(h * CH, CH), pl.ds(c0, DC)],
            bufs[h], sems[h]).wait()
        pltpu.make_async_copy(tgt_hbm.at[b, pl.ds(h * CH, CH)],
                              tbufs[h], sems[h]).wait()

    def zero_partial_accs():
        @plsc.parallel_loop(0, C, unroll=4)
        def zloop(c):
            acc_pm[c] = zvec
            acc_pm2[c] = zvec
            acc_l2[c] = zvec

        @plsc.parallel_loop(0, C // L, unroll=2)
        def zcnt(g):
            cnt[pl.ds(g * L, L)] = zvec

    def zero_accp():
        @plsc.parallel_loop(0, C, unroll=4)
        def zp(c):
            acc_p[c] = zvec

    def zero_lpsum():
        @plsc.parallel_loop(0, C // L, unroll=2)
        def zl(g):
            lpsum[pl.ds(g * L, L)] = zvec

    def fire_out(b):
        pltpu.async_copy(acc_p, pp_hbm.at[b, :, pl.ds(c0, DC)], semo)
        pltpu.async_copy(pmout, pm_hbm.at[b, :, pl.ds(c0, DC)], semo)
        pltpu.async_copy(lpsum, part_hbm.at[wid, b], semo)

    def drain_out(b):
        pltpu.make_async_copy(acc_p, pp_hbm.at[b, :, pl.ds(c0, DC)],
                              semo).wait()
        pltpu.make_async_copy(pmout, pm_hbm.at[b, :, pl.ds(c0, DC)],
                              semo).wait()
        pltpu.make_async_copy(lpsum, part_hbm.at[wid, b], semo).wait()

    def do_half(b, h):
        if h == 0:
            fire(b, 1)
        else:
            @pl.when(b < B - 1)
            def _():
                fire(b + 1, 0)

        drain(b, h)
        buf = bufs[h]
        tbuf = tbufs[h]

        @plsc.parallel_loop(0, CH // L, unroll=4)
        def count(g):
            tvec = tbuf[pl.ds(g * L, L)]
            plsc.addupdate_scatter(cnt, [tvec], onevec)

        @plsc.parallel_loop(0, CH, unroll=8)
        def row(n):
            tb = plsc.load_gather(tbuf, [jnp.broadcast_to(n, (L,))])
            bits = plsc.bitcast(buf[n], jnp.int32)
            p = plsc.bitcast(bits & jnp.int32(-65536), jnp.float32)
            m = plsc.bitcast(lax.shift_left(bits, 16), jnp.float32)
            pm = p * m
            pmm = pm * m
            l2 = _log2_biased(p)
            plsc.addupdate_scatter(acc_p, [tb, iota], p)
            plsc.addupdate_scatter(acc_pm, [tb, iota], pm)
            plsc.addupdate_scatter(acc_pm2, [tb, iota], pmm)
            plsc.addupdate_scatter(acc_l2, [tb, iota], l2)

    def batch_body(b, carry):
        do_half(b, 0)
        do_half(b, 1)

        @pl.when(b > 0)
        def _():
            pltpu.make_async_copy(pmout, pm_hbm.at[b - 1, :, pl.ds(c0, DC)],
                                  semo).wait()
            pltpu.make_async_copy(lpsum, part_hbm.at[wid, b - 1], semo).wait()

        zero_lpsum()

        @plsc.parallel_loop(0, C, unroll=4)
        def cflush(c):
            s1 = acc_p[c]
            s2 = acc_pm[c]
            s3 = acc_pm2[c]
            sl = acc_l2[c]
            nv = plsc.load_gather(cnt, [jnp.broadcast_to(c, (L,))])
            pmv = s2 / s1
            pmout[c] = pmv
            l2s1 = _log2_hi(s1)
            nm = jnp.maximum(nv, onevec)
            lp = 0.5 * ((1.0 - nm) * jnp.float32(LOG2PI)
                        + jnp.float32(LN2) * (sl - l2s1)
                        + (s1 * pmv * pmv - s3))
            plsc.addupdate_scatter(lpsum, [jnp.broadcast_to(c, (L,))], lp)

        pltpu.async_copy(acc_p, pp_hbm.at[b, :, pl.ds(c0, DC)], semo)
        pltpu.async_copy(pmout, pm_hbm.at[b, :, pl.ds(c0, DC)], semo)
        pltpu.async_copy(lpsum, part_hbm.at[wid, b], semo)
        zero_partial_accs()
        pltpu.make_async_copy(acc_p, pp_hbm.at[b, :, pl.ds(c0, DC)],
                              semo).wait()
        zero_accp()
        return carry

    zero_partial_accs()
    zero_accp()
    zero_lpsum()
    fire(0, 0)
    lax.fori_loop(0, B, batch_body, 0)
    pltpu.make_async_copy(pmout, pm_hbm.at[B - 1, :, pl.ds(c0, DC)],
                          semo).wait()
    pltpu.make_async_copy(lpsum, part_hbm.at[wid, B - 1], semo).wait()


@jax.jit
def kernel(means, precisions, targets):
    mesh = plsc.VectorSubcoreMesh(core_axis_name="c", subcore_axis_name="s",
                                  num_cores=NC, num_subcores=NS)
    k = pl.kernel(
        _body,
        out_type=(
            jax.ShapeDtypeStruct((B, C, D), jnp.float32),
            jax.ShapeDtypeStruct((B, C, D), jnp.float32),
            jax.ShapeDtypeStruct((NW, B, C), jnp.float32),
        ),
        mesh=mesh,
        compiler_params=pltpu.CompilerParams(use_tc_tiling_on_sc=False,
                                             needs_layout_passes=False),
        scratch_types=[
            pltpu.VMEM((CH, DC), jnp.float32),
            pltpu.VMEM((CH, DC), jnp.float32),
            pltpu.VMEM((CH,), jnp.int32),
            pltpu.VMEM((CH,), jnp.int32),
            pltpu.VMEM((C, L), jnp.float32),
            pltpu.VMEM((C, L), jnp.float32),
            pltpu.VMEM((C, L), jnp.float32),
            pltpu.VMEM((C, L), jnp.float32),
            pltpu.VMEM((C,), jnp.float32),
            pltpu.VMEM((C, L), jnp.float32),
            pltpu.VMEM((C,), jnp.float32),
            pltpu.SemaphoreType.DMA,
            pltpu.SemaphoreType.DMA,
            pltpu.SemaphoreType.DMA,
        ],
    )
    pbits = lax.bitcast_convert_type(precisions.astype(jnp.bfloat16),
                                     jnp.uint16).astype(jnp.uint32) << 16
    mbits = lax.bitcast_convert_type(means.astype(jnp.bfloat16),
                                     jnp.uint16).astype(jnp.uint32)
    mp = lax.bitcast_convert_type(pbits | mbits, jnp.float32)
    pm, pp, part = k(mp, targets)
    lpn = part.sum(axis=0)
    return (pm, pp, lpn)


# speedup vs baseline: 2.0469x; 1.0491x over previous
import functools
import math

import jax
import jax.numpy as jnp
from jax import lax
from jax.experimental import pallas as pl
from jax.experimental.pallas import tpu as pltpu
from jax.experimental.pallas import tpu_sc as plsc

B, N, D, C = 16, 2048, 512, 128
NC, NS, L = 2, 16, 16
NW = NC * NS
DC = D // NW
CH = N // 2

LN2 = 0.6931471805599453
LOG2PI = 1.8378770664093453

_R0 = 0.0008254628229340533
_R1 = 1.415653190432736
_R2 = -0.5687040530057521
_R3 = 0.15270028479752185

_C0 = 2.443438720245439e-06
_C1 = 1.4424535262105997
_C2 = -0.7173127802648079
_C3 = 0.454508492199418
_C4 = -0.2726975648521658
_C5 = 0.1176130840660221
_C6 = -0.024568534745087942

_S0 = _R0 - 127.0
_S1 = _R1 * 2.0 ** -23
_S2 = _R2 * 2.0 ** -46
_S3 = _R3 * 2.0 ** -69


def _log2_biased(x):
    bits = plsc.bitcast(x, jnp.int32)
    ebias = lax.shift_right_logical(bits, 23).astype(jnp.float32)
    zm = (bits & 0x007FFFFF).astype(jnp.float32)
    poly = ((jnp.float32(_S3) * zm + jnp.float32(_S2)) * zm
            + jnp.float32(_S1)) * zm + jnp.float32(_S0)
    return poly + ebias


def _log2_hi(x):
    bits = plsc.bitcast(x, jnp.int32)
    ebias = lax.shift_right_logical(bits, 23).astype(jnp.float32)
    fbits = (bits & 0x007FFFFF) | 0x3F800000
    z = plsc.bitcast(fbits, jnp.float32) - 1.0
    acc = jnp.float32(_C6)
    for c in (_C5, _C4, _C3, _C2, _C1, _C0):
        acc = acc * z + jnp.float32(c)
    return acc + (ebias - 127.0)


def _body(means_hbm, prec_hbm, tgt_hbm, pm_hbm, pp_hbm, part_hbm,
          pbuf0, pbuf1, mbuf0, mbuf1, tbuf0, tbuf1,
          acc_p, acc_pm, acc_pm2, acc_l2, cnt, pmout, lpsum,
          sem0, sem1, semo):
    cid = lax.axis_index("c")
    sid = lax.axis_index("s")
    wid = sid * NC + cid
    c0 = wid * DC

    pbufs = (pbuf0, pbuf1)
    mbufs = (mbuf0, mbuf1)
    tbufs = (tbuf0, tbuf1)
    sems = (sem0, sem1)

    zvec = jnp.zeros((L,), jnp.float32)
    onevec = jnp.ones((L,), jnp.float32)
    iota = lax.iota(jnp.int32, L)

    def fire(b, h):
        pltpu.async_copy(prec_hbm.at[b, pl.ds(h * CH, CH), pl.ds(c0, DC)],
                         pbufs[h], sems[h])
        pltpu.async_copy(means_hbm.at[b, pl.ds(h * CH, CH), pl.ds(c0, DC)],
                         mbufs[h], sems[h])
        pltpu.async_copy(tgt_hbm.at[b, pl.ds(h * CH, CH)], tbufs[h], sems[h])

    def drain(b, h):
        pltpu.make_async_copy(
            prec_hbm.at[b, pl.ds(h * CH, CH), pl.ds(c0, DC)],
            pbufs[h], sems[h]).wait()
        pltpu.make_async_copy(
            means_hbm.at[b, pl.ds(h * CH, CH), pl.ds(c0, DC)],
            mbufs[h], sems[h]).wait()
        pltpu.make_async_copy(tgt_hbm.at[b, pl.ds(h * CH, CH)],
                              tbufs[h], sems[h]).wait()

    def zero_partial_accs():
        @plsc.parallel_loop(0, C, unroll=4)
        def zloop(c):
            acc_pm[c] = zvec
            acc_pm2[c] = zvec
            acc_l2[c] = zvec

        @plsc.parallel_loop(0, C // L, unroll=2)
        def zcnt(g):
            cnt[pl.ds(g * L, L)] = zvec

    def zero_accp():
        @plsc.parallel_loop(0, C, unroll=4)
        def zp(c):
            acc_p[c] = zvec

    def zero_lpsum():
        @plsc.parallel_loop(0, C // L, unroll=2)
        def zl(g):
            lpsum[pl.ds(g * L, L)] = zvec

    def do_half(b, h):
        if h == 0:
            fire(b, 1)
        else:
            @pl.when(b < B - 1)
            def _():
                fire(b + 1, 0)

        drain(b, h)
        pbuf = pbufs[h]
        mbuf = mbufs[h]
        tbuf = tbufs[h]

        @plsc.parallel_loop(0, CH // L, unroll=4)
        def count(g):
            tvec = tbuf[pl.ds(g * L, L)]
            plsc.addupdate_scatter(cnt, [tvec], onevec)

        @plsc.parallel_loop(0, CH, unroll=8)
        def row(n):
            tb = plsc.load_gather(tbuf, [jnp.broadcast_to(n, (L,))])
            p = pbuf[n]
            m = mbuf[n]
            pm = p * m
            pmm = pm * m
            l2 = _log2_biased(p)
            plsc.addupdate_scatter(acc_p, [tb, iota], p)
            plsc.addupdate_scatter(acc_pm, [tb, iota], pm)
            plsc.addupdate_scatter(acc_pm2, [tb, iota], pmm)
            plsc.addupdate_scatter(acc_l2, [tb, iota], l2)

    def batch_body(b, carry):
        do_half(b, 0)
        do_half(b, 1)

        @pl.when(b > 0)
        def _():
            pltpu.make_async_copy(pmout, pm_hbm.at[b - 1, :, pl.ds(c0, DC)],
                                  semo).wait()
            pltpu.make_async_copy(lpsum, part_hbm.at[wid, b - 1], semo).wait()

        zero_lpsum()

        @plsc.parallel_loop(0, C, unroll=4)
        def cflush(c):
            s1 = acc_p[c]
            s2 = acc_pm[c]
            s3 = acc_pm2[c]
            sl = acc_l2[c]
            nv = plsc.load_gather(cnt, [jnp.broadcast_to(c, (L,))])
            pmv = s2 / s1
            pmout[c] = pmv
            l2s1 = _log2_hi(s1)
            nm = jnp.maximum(nv, onevec)
            lp = 0.5 * ((1.0 - nm) * jnp.float32(LOG2PI)
                        + jnp.float32(LN2) * (sl - l2s1)
                        + (s1 * pmv * pmv - s3))
            plsc.addupdate_scatter(lpsum, [jnp.broadcast_to(c, (L,))], lp)

        pltpu.async_copy(acc_p, pp_hbm.at[b, :, pl.ds(c0, DC)], semo)
        pltpu.async_copy(pmout, pm_hbm.at[b, :, pl.ds(c0, DC)], semo)
        pltpu.async_copy(lpsum, part_hbm.at[wid, b], semo)
        zero_partial_accs()
        pltpu.make_async_copy(acc_p, pp_hbm.at[b, :, pl.ds(c0, DC)],
                              semo).wait()
        zero_accp()
        return carry

    zero_partial_accs()
    zero_accp()
    zero_lpsum()
    fire(0, 0)
    lax.fori_loop(0, B, batch_body, 0)
    pltpu.make_async_copy(pmout, pm_hbm.at[B - 1, :, pl.ds(c0, DC)],
                          semo).wait()
    pltpu.make_async_copy(lpsum, part_hbm.at[wid, B - 1], semo).wait()


@jax.jit
def kernel(means, precisions, targets):
    mesh = plsc.VectorSubcoreMesh(core_axis_name="c", subcore_axis_name="s",
                                  num_cores=NC, num_subcores=NS)
    k = pl.kernel(
        _body,
        out_type=(
            jax.ShapeDtypeStruct((B, C, D), jnp.float32),
            jax.ShapeDtypeStruct((B, C, D), jnp.float32),
            jax.ShapeDtypeStruct((NW, B, C), jnp.float32),
        ),
        mesh=mesh,
        compiler_params=pltpu.CompilerParams(use_tc_tiling_on_sc=False,
                                             needs_layout_passes=False),
        scratch_types=[
            pltpu.VMEM((CH, DC), jnp.float32),
            pltpu.VMEM((CH, DC), jnp.float32),
            pltpu.VMEM((CH, DC), jnp.float32),
            pltpu.VMEM((CH, DC), jnp.float32),
            pltpu.VMEM((CH,), jnp.int32),
            pltpu.VMEM((CH,), jnp.int32),
            pltpu.VMEM((C, L), jnp.float32),
            pltpu.VMEM((C, L), jnp.float32),
            pltpu.VMEM((C, L), jnp.float32),
            pltpu.VMEM((C, L), jnp.float32),
            pltpu.VMEM((C,), jnp.float32),
            pltpu.VMEM((C, L), jnp.float32),
            pltpu.VMEM((C,), jnp.float32),
            pltpu.SemaphoreType.DMA,
            pltpu.SemaphoreType.DMA,
            pltpu.SemaphoreType.DMA,
        ],
    )
    pm, pp, part = k(means, precisions, targets)
    lpn = part.sum(axis=0)
    return (pm, pp, lpn)
